# Initial kernel scaffold; baseline (speedup 1.0000x reference)
import jax
import jax.numpy as jnp
from jax.experimental import pallas as pl


def kernel(boxes, objectness, breed_conf):
    raise NotImplementedError("write your pallas kernel here")


# speedup vs baseline: 27.9493x; 27.9493x over previous
import jax
import jax.numpy as jnp
from jax import lax
from jax.experimental import pallas as pl
from jax.experimental.pallas import tpu as pltpu

N = 20000
MAX_DET = 300
CONF_THRES = 0.25
IOU_THRES = 0.5

_ROWS = 160
_P = _ROWS * 128
_OUT_ROWS = 304


def _nms_body(x1r, y1r, x2r, y2r, obr, brr, outr, scr, arr):
    neg_inf = jnp.float32(-jnp.inf)
    x1 = x1r[...]
    y1 = y1r[...]
    x2 = x2r[...]
    y2 = y2r[...]
    obj = jnp.clip(obr[...], 0.0, 1.0)
    br = jnp.clip(brr[...], 0.0, 1.0)
    s = obj * br
    scr[...] = jnp.where(s >= CONF_THRES, s, neg_inf)
    arr[...] = (x2 - x1) * (y2 - y1)
    outr[...] = jnp.zeros((_OUT_ROWS, 128), jnp.float32)

    idx = (lax.broadcasted_iota(jnp.int32, (_ROWS, 128), 0) * 128
           + lax.broadcasted_iota(jnp.int32, (_ROWS, 128), 1))
    lane = lax.broadcasted_iota(jnp.int32, (1, 128), 1)

    def it(i, carry):
        sc = scr[...]
        m = jnp.max(sc)
        win = jnp.min(jnp.where(sc == m, idx, jnp.int32(_P)))
        ok = m > neg_inf
        r = win // 128
        c = win % 128
        onehot = lane == c
        wx1 = jnp.max(jnp.where(onehot, x1r[pl.ds(r, 1), :], -1e30))
        wy1 = jnp.max(jnp.where(onehot, y1r[pl.ds(r, 1), :], -1e30))
        wx2 = jnp.max(jnp.where(onehot, x2r[pl.ds(r, 1), :], -1e30))
        wy2 = jnp.max(jnp.where(onehot, y2r[pl.ds(r, 1), :], -1e30))
        wx1 = jnp.where(ok, wx1, jnp.float32(2e9))
        wy1 = jnp.where(ok, wy1, jnp.float32(2e9))
        wx2 = jnp.where(ok, wx2, jnp.float32(-2e9))
        wy2 = jnp.where(ok, wy2, jnp.float32(-2e9))
        wa = (wx2 - wx1) * (wy2 - wy1)
        xx1 = jnp.maximum(wx1, x1)
        yy1 = jnp.maximum(wy1, y1)
        xx2 = jnp.minimum(wx2, x2)
        yy2 = jnp.minimum(wy2, y2)
        inter = jnp.maximum(xx2 - xx1, 0.0) * jnp.maximum(yy2 - yy1, 0.0)
        union = wa + arr[...] - inter
        iou = inter / jnp.maximum(union, 1e-9)
        scr[...] = jnp.where(iou >= IOU_THRES, neg_inf, sc)
        mval = jnp.where(ok, m, 0.0)
        okf = jnp.where(ok, 1.0, 0.0)
        row = jnp.where(lane == 0, wx1,
              jnp.where(lane == 1, wy1,
              jnp.where(lane == 2, wx2,
              jnp.where(lane == 3, wy2,
              jnp.where(lane == 4, mval, 0.0)))))
        outr[pl.ds(i, 1), :] = row * okf
        return carry

    lax.fori_loop(0, MAX_DET, it, 0)


def kernel(boxes, objectness, breed_conf):
    pad = _P - N
    x1 = jnp.pad(boxes[:, 0], (0, pad)).reshape(_ROWS, 128)
    y1 = jnp.pad(boxes[:, 1], (0, pad)).reshape(_ROWS, 128)
    x2 = jnp.pad(boxes[:, 2], (0, pad)).reshape(_ROWS, 128)
    y2 = jnp.pad(boxes[:, 3], (0, pad)).reshape(_ROWS, 128)
    ob = jnp.pad(objectness, (0, pad)).reshape(_ROWS, 128)
    br = jnp.pad(breed_conf, (0, pad)).reshape(_ROWS, 128)
    out = pl.pallas_call(
        _nms_body,
        out_shape=jax.ShapeDtypeStruct((_OUT_ROWS, 128), jnp.float32),
        in_specs=[pl.BlockSpec(memory_space=pltpu.VMEM)] * 6,
        out_specs=pl.BlockSpec(memory_space=pltpu.VMEM),
        scratch_shapes=[
            pltpu.VMEM((_ROWS, 128), jnp.float32),
            pltpu.VMEM((_ROWS, 128), jnp.float32),
        ],
    )(x1, y1, x2, y2, ob, br)
    return out[:MAX_DET, :5]
